# baseline (speedup 1.0000x reference)
import functools

import jax
import jax.numpy as jnp
from jax import lax
from jax.experimental import pallas as pl
from jax.experimental.pallas import tpu as pltpu
from jax.experimental.pallas import tpu_sc as plsc

NC = 2
NS = 16
NW = NC * NS
LANES = 16


def _build(n_children, n_parents, dim, row_block, win_rows, interpret=False):
    assert dim % LANES == 0
    nd = dim // LANES
    assert n_children % row_block == 0 and row_block % LANES == 0
    assert n_parents >= win_rows
    n_groups = row_block // LANES
    ppw = -(-n_parents // (NW * win_rows)) * win_rows
    p_real = n_parents
    win_align = 8 if (win_rows % 8 == 0 and ppw % 8 == 0
                      and (p_real - win_rows) % 8 == 0) else 1

    mesh = plsc.VectorSubcoreMesh(
        core_axis_name="c", subcore_axis_name="s",
        num_cores=NC, num_subcores=NS,
    )

    @functools.partial(
        pl.kernel,
        out_type=jax.ShapeDtypeStruct((NW * LANES,), jnp.float32),
        mesh=mesh,
        scratch_types=[
            pltpu.VMEM((row_block, dim), jnp.float32),
            pltpu.VMEM((row_block,), jnp.int32),
            pltpu.VMEM((win_rows, dim), jnp.float32),
            pltpu.VMEM((LANES,), jnp.int32),
            pltpu.VMEM((dim,), jnp.float32),
            pltpu.VMEM((LANES,), jnp.float32),
        ],
        interpret=interpret,
    )
    def sc_kernel(child_hbm, idx_hbm, parent_hbm, cuts_hbm, out_hbm,
                  child_v, idx_v, win_v, cuts_v, mark_v, loss_v):
        w = lax.axis_index("s") * NC + lax.axis_index("c")
        plo = w * ppw
        phi = jnp.minimum(plo + ppw, p_real)

        pltpu.sync_copy(
            cuts_hbm.at[pl.ds(pl.multiple_of(w * LANES, LANES), LANES)],
            cuts_v)
        cvec = cuts_v[...]
        s_lo = cvec[0]
        s_hi = cvec[1]

        zeros = jnp.zeros((LANES,), jnp.float32)
        loss_v[...] = zeros
        for k in range(nd):
            mark_v[pl.ds(k * LANES, LANES)] = zeros

        def load_window(wlo):
            start = pl.multiple_of(jnp.minimum(wlo, p_real - win_rows),
                                   win_align)
            pltpu.sync_copy(parent_hbm.at[pl.ds(start, win_rows), :], win_v)

            def yrow(r, lv):
                for k in range(nd):
                    t = win_v[r, pl.ds(k * LANES, LANES)]
                    lv += t * t
                return lv

            lv = lax.fori_loop(wlo - start, win_rows, yrow, zeros)
            loss_v[...] = loss_v[...] + lv

        load_window(plo)

        def emit(cur, cnt, accs, wlo):
            nadv = (cur - wlo) // win_rows

            def wbody(i, wl):
                wl2 = wl + win_rows
                load_window(wl2)
                return wl2

            wlo = lax.fori_loop(0, nadv, wbody, wlo)
            start = pl.multiple_of(jnp.minimum(wlo, p_real - win_rows),
                                   win_align)
            rrow = cur - start
            cntv = jnp.full((LANES,), cnt.astype(jnp.float32))
            inv = jnp.ones((LANES,), jnp.float32) / cntv
            lv = zeros
            for k in range(nd):
                seg = accs[k] - mark_v[pl.ds(k * LANES, LANES)]
                m = seg * inv
                y = win_v[rrow, pl.ds(k * LANES, LANES)]
                lv += m * (m - (y + y))
            loss_v[...] = loss_v[...] + lv
            return wlo

        def row_step(q, gpos, rr, carry):
            accs = carry[:nd]
            cur, spos, wlo = carry[nd], carry[nd + 1], carry[nd + 2]
            row = tuple(child_v[rr, pl.ds(k * LANES, LANES)]
                        for k in range(nd))

            def boundary(op):
                cur, spos, wlo = op

                def do_emit(wl):
                    return emit(cur, gpos - spos, accs, wl)

                wlo = lax.cond(cur >= 0, do_emit, lambda wl: wl, wlo)
                for k in range(nd):
                    mark_v[pl.ds(k * LANES, LANES)] = accs[k]
                return q, gpos, wlo

            cur, spos, wlo = lax.cond(q != cur, boundary, lambda op: op,
                                      (cur, spos, wlo))
            accs = tuple(a + x for a, x in zip(accs, row))
            return accs + (cur, spos, wlo)

        def block_body(b, carry):
            base = pl.multiple_of(b * row_block, row_block)
            pltpu.sync_copy(child_hbm.at[pl.ds(base, row_block), :], child_v)
            pltpu.sync_copy(idx_hbm.at[pl.ds(base, row_block)], idx_v)

            def group_body(g, gcarry):
                qraw = idx_v[pl.ds(g * LANES, LANES)]
                gvec = base + g * LANES + lax.iota(jnp.int32, LANES)
                qv = jnp.where(gvec < s_lo, jnp.int32(-1),
                               jnp.where(gvec >= s_hi, jnp.int32(-2), qraw))
                for r_in in range(LANES):
                    gcarry = row_step(qv[r_in], base + g * LANES + r_in,
                                      g * LANES + r_in, gcarry)
                return gcarry

            return lax.fori_loop(0, n_groups, group_body, carry)

        zero_acc = tuple(zeros for _ in range(nd))
        init = zero_acc + (jnp.int32(-1), jnp.int32(0), plo)
        b0 = s_lo // row_block
        b1 = (s_hi + row_block - 1) // row_block
        carry = lax.fori_loop(b0, b1, block_body, init)
        accs = carry[:nd]
        cur, spos, wlo = carry[nd], carry[nd + 1], carry[nd + 2]

        def do_fin(wl):
            return emit(cur, s_hi - spos, accs, wl)

        wlo = lax.cond(cur >= 0, do_fin, lambda wl: wl, wlo)

        ndrain = (phi - wlo - 1) // win_rows

        def dbody(i, wl):
            wl2 = wl + win_rows
            load_window(wl2)
            return wl2

        wlo = lax.fori_loop(0, ndrain, dbody, wlo)

        pltpu.sync_copy(
            loss_v,
            out_hbm.at[pl.ds(pl.multiple_of(w * LANES, LANES), LANES)])

    return sc_kernel, ppw


def _make_cuts(child_to_parent_idx, ppw):
    bounds = jnp.arange(NW + 1, dtype=jnp.int32) * ppw
    cuts = jnp.searchsorted(child_to_parent_idx, bounds, side="left")
    cuts = cuts.astype(jnp.int32)
    rows = jnp.stack([cuts[:-1], cuts[1:]], axis=1)
    return jnp.pad(rows, ((0, 0), (0, LANES - 2))).reshape(-1)


def kernel(child_state, parent_state, child_to_parent_idx, num_parents):
    n_children, dim = child_state.shape
    n_parents = parent_state.shape[0]
    sc_kernel, ppw = _build(n_children, n_parents, dim,
                            row_block=128, win_rows=64)
    cuts = _make_cuts(child_to_parent_idx, ppw)
    partials = sc_kernel(child_state, child_to_parent_idx, parent_state, cuts)
    total = jnp.sum(partials)
    return total / jnp.float32(n_parents * dim)
